# baseline (speedup 1.0000x reference)
import functools

import jax
import jax.numpy as jnp
from jax import lax
from jax.experimental import pallas as pl
from jax.experimental.pallas import tpu as pltpu
from jax.experimental.pallas import tpu_sc as plsc

N = 10000
D = 128
E = 320000

NC = 2
NS = 16
NW = NC * NS

EPT = E // NW
CH = 128
NIT = EPT // CH
TCH = EPT - NIT * CH
NBUF = 2
NIB = 6
RPT = 624
TAIL = N - NS * RPT
ZCH = 78

_mesh = plsc.VectorSubcoreMesh(core_axis_name="c", subcore_axis_name="s")


@functools.partial(
    pl.kernel,
    mesh=_mesh,
    out_type=jax.ShapeDtypeStruct((NC, N, D), jnp.float32),
    scratch_types=(
        [pltpu.VMEM((CH,), jnp.int32) for _ in range(NIB)]
        + [pltpu.VMEM((CH,), jnp.int32) for _ in range(NIB)]
        + [pltpu.VMEM((TCH,), jnp.int32) for _ in range(2)]
        + [pltpu.VMEM((CH, D), jnp.float32) for _ in range(NBUF)]
        + [pltpu.VMEM_SHARED((N, D), jnp.float32)]
        + [pltpu.SemaphoreType.DMA for _ in range(NIB + 2 * NBUF)]
    ),
)
def _edge_scatter(x_hbm, esrc_hbm, edst_hbm, out_hbm, *rest):
    sidxb = rest[0:NIB]
    didxb = rest[NIB:2 * NIB]
    tsidx, tdidx = rest[2 * NIB:2 * NIB + 2]
    bufs = rest[2 * NIB + 2:2 * NIB + 2 + NBUF]
    acc = rest[2 * NIB + 2 + NBUF]
    semi = rest[2 * NIB + 3 + NBUF:2 * NIB + 3 + NBUF + NIB]
    semg = rest[2 * NIB + 3 + NBUF + NIB:2 * NIB + 3 + NBUF + NIB + NBUF]
    sems = rest[2 * NIB + 3 + NBUF + NIB + NBUF:]
    r0 = bufs[0]
    c = lax.axis_index("c")
    s = lax.axis_index("s")
    w = c * NS + s
    ebase = w * EPT

    @pl.loop(0, ZCH)
    def _(r):
        for j in range(D // 16):
            r0[r, pl.ds(j * 16, 16)] = jnp.zeros((16,), jnp.float32)

    for k in range(RPT // ZCH):
        pltpu.async_copy(r0.at[pl.ds(0, ZCH)], acc.at[pl.ds(s * RPT + k * ZCH, ZCH)],
                         semi[0])

    @pl.when(s == 0)
    def _():
        pltpu.async_copy(r0.at[pl.ds(0, TAIL)], acc.at[pl.ds(NS * RPT, TAIL)], semi[0])

    for k in range(RPT // ZCH):
        pltpu.make_async_copy(r0.at[pl.ds(0, ZCH)],
                              acc.at[pl.ds(s * RPT + k * ZCH, ZCH)], semi[0]).wait()

    @pl.when(s == 0)
    def _():
        pltpu.make_async_copy(r0.at[pl.ds(0, TAIL)],
                              acc.at[pl.ds(NS * RPT, TAIL)], semi[0]).wait()

    plsc.subcore_barrier()

    def start_idx(i, j):
        pltpu.async_copy(esrc_hbm.at[pl.ds(ebase + i * CH, CH)], sidxb[j], semi[j])
        pltpu.async_copy(edst_hbm.at[pl.ds(ebase + i * CH, CH)], didxb[j], semi[j])

    def wait_idx(j):
        pltpu.make_async_copy(esrc_hbm.at[pl.ds(0, CH)], sidxb[j], semi[j]).wait()
        pltpu.make_async_copy(edst_hbm.at[pl.ds(0, CH)], didxb[j], semi[j]).wait()

    for j in range(NIB):
        start_idx(j, j)
    wait_idx(0)
    pltpu.async_copy(x_hbm.at[sidxb[0]], bufs[0], semg[0])

    @pl.loop(0, NIT, step=NIB)
    def _(g):
        for b6 in range(NIB):
            bb = b6 % NBUF
            i = g + b6
            pltpu.make_async_copy(x_hbm.at[sidxb[b6]], bufs[bb], semg[bb]).wait()
            pltpu.async_copy(bufs[bb], acc.at[didxb[b6]], sems[bb], add=True)

            @pl.when(i >= 1)
            def _():
                pltpu.make_async_copy(bufs[1 - bb], acc.at[didxb[b6]],
                                      sems[1 - bb]).wait()
                jm = (b6 - 1) % NIB

                @pl.when(i + NIB - 1 < NIT)
                def _():
                    pltpu.async_copy(
                        esrc_hbm.at[pl.ds(ebase + (i + NIB - 1) * CH, CH)],
                        sidxb[jm], semi[jm])
                    pltpu.async_copy(
                        edst_hbm.at[pl.ds(ebase + (i + NIB - 1) * CH, CH)],
                        didxb[jm], semi[jm])

            @pl.when(i + 1 < NIT)
            def _():
                j1 = (b6 + 1) % NIB
                wait_idx(j1)
                pltpu.async_copy(x_hbm.at[sidxb[j1]], bufs[1 - bb], semg[1 - bb])

    lb = (NIT - 1) % NBUF
    pltpu.make_async_copy(bufs[lb], acc.at[didxb[(NIT - 1) % NIB]], sems[lb]).wait()

    pltpu.sync_copy(esrc_hbm.at[pl.ds(ebase + NIT * CH, TCH)], tsidx)
    pltpu.sync_copy(edst_hbm.at[pl.ds(ebase + NIT * CH, TCH)], tdidx)
    pltpu.async_copy(x_hbm.at[tsidx], bufs[0].at[pl.ds(0, TCH)], semg[0])
    pltpu.make_async_copy(x_hbm.at[tsidx], bufs[0].at[pl.ds(0, TCH)], semg[0]).wait()
    pltpu.sync_copy(bufs[0].at[pl.ds(0, TCH)], acc.at[tdidx], add=True)

    plsc.subcore_barrier()

    pltpu.sync_copy(acc.at[pl.ds(s * RPT, RPT)], out_hbm.at[c, pl.ds(s * RPT, RPT)])

    @pl.when(s == 0)
    def _():
        pltpu.sync_copy(acc.at[pl.ds(NS * RPT, TAIL)], out_hbm.at[c, pl.ds(NS * RPT, TAIL)])


_BLK = 400


def _mm_relu_body(p_ref, w_ref, bias_ref, o_ref):
    sm = p_ref[0] + p_ref[1]
    o_ref[...] = jnp.maximum(
        jnp.dot(sm, w_ref[...], preferred_element_type=jnp.float32) + bias_ref[...],
        0.0,
    )


def _mm_relu(p, w, bias):
    grid = (N // _BLK,)
    return pl.pallas_call(
        _mm_relu_body,
        grid=grid,
        in_specs=[
            pl.BlockSpec((NC, _BLK, D), lambda i: (0, i, 0)),
            pl.BlockSpec((D, D), lambda i: (0, 0)),
            pl.BlockSpec((1, D), lambda i: (0, 0)),
        ],
        out_specs=pl.BlockSpec((_BLK, D), lambda i: (i, 0)),
        out_shape=jax.ShapeDtypeStruct((N, D), jnp.float32),
    )(p, w, bias)


def _mm_final_body(p_ref, w2_ref, b2_ref, w3_ref, b3_ref,
                   emb_ref, ge_ref, acc_ref):
    i = pl.program_id(0)
    sm = p_ref[0] + p_ref[1]
    emb_ref[...] = (
        jnp.dot(sm, w2_ref[...], preferred_element_type=jnp.float32) + b2_ref[...]
    )
    csum = jnp.sum(sm, axis=0, keepdims=True)

    @pl.when(i == 0)
    def _():
        acc_ref[...] = csum

    @pl.when(i > 0)
    def _():
        acc_ref[...] = acc_ref[...] + csum

    @pl.when(i == pl.num_programs(0) - 1)
    def _():
        ge_ref[...] = (
            jnp.dot(acc_ref[...], w3_ref[...], preferred_element_type=jnp.float32)
            + float(N) * b3_ref[...]
        )


def _mm_final(p, w2, b2, w3, b3):
    grid = (N // _BLK,)
    return pl.pallas_call(
        _mm_final_body,
        grid=grid,
        in_specs=[
            pl.BlockSpec((NC, _BLK, D), lambda i: (0, i, 0)),
            pl.BlockSpec((D, D), lambda i: (0, 0)),
            pl.BlockSpec((1, D), lambda i: (0, 0)),
            pl.BlockSpec((D, D), lambda i: (0, 0)),
            pl.BlockSpec((1, D), lambda i: (0, 0)),
        ],
        out_specs=[
            pl.BlockSpec((_BLK, D), lambda i: (i, 0)),
            pl.BlockSpec((1, D), lambda i: (0, 0)),
        ],
        out_shape=[
            jax.ShapeDtypeStruct((N, D), jnp.float32),
            jax.ShapeDtypeStruct((1, D), jnp.float32),
        ],
        scratch_shapes=[pltpu.VMEM((1, D), jnp.float32)],
    )(p, w2, b2, w3, b3)


def kernel(x, edge_index, W1, b1, W2, b2, W3, b3):
    esrc = edge_index[0]
    edst = edge_index[1]

    p1 = _edge_scatter(x, esrc, edst)
    h = _mm_relu(p1, W1, b1.reshape(1, D))
    p2 = _edge_scatter(h, esrc, edst)
    embed, graph_embed = _mm_final(
        p2, W2, b2.reshape(1, D), W3, b3.reshape(1, D)
    )
    return (embed, graph_embed)


# speedup vs baseline: 1.2281x; 1.2281x over previous
import functools

import jax
import jax.numpy as jnp
from jax import lax
from jax.experimental import pallas as pl
from jax.experimental.pallas import tpu as pltpu
from jax.experimental.pallas import tpu_sc as plsc

N = 10000
D = 128
E = 320000

NC = 2
NS = 16
NW = NC * NS

EPT = E // NW
CH = 80
NIT = EPT // CH
NBUF = 3
NIB = 6
RPT = 624
TAIL = N - NS * RPT

_mesh = plsc.VectorSubcoreMesh(core_axis_name="c", subcore_axis_name="s")


@functools.partial(
    pl.kernel,
    mesh=_mesh,
    out_type=jax.ShapeDtypeStruct((NC, N, D), jnp.float32),
    scratch_types=(
        [pltpu.VMEM((CH,), jnp.int32) for _ in range(NIB)]
        + [pltpu.VMEM((CH,), jnp.int32) for _ in range(NIB)]
        + [pltpu.VMEM((CH, D), jnp.float32) for _ in range(NBUF)]
        + [pltpu.VMEM_SHARED((N, D), jnp.float32)]
        + [pltpu.SemaphoreType.DMA for _ in range(NIB + 2 * NBUF)]
    ),
)
def _edge_scatter(x_hbm, esrc_hbm, edst_hbm, out_hbm, *rest):
    sidxb = rest[0:NIB]
    didxb = rest[NIB:2 * NIB]
    bufs = rest[2 * NIB:2 * NIB + NBUF]
    acc = rest[2 * NIB + NBUF]
    semi = rest[2 * NIB + NBUF + 1:2 * NIB + NBUF + 1 + NIB]
    semg = rest[2 * NIB + NBUF + 1 + NIB:2 * NIB + NBUF + 1 + NIB + NBUF]
    sems = rest[2 * NIB + NBUF + 1 + NIB + NBUF:]
    r0 = bufs[0]
    c = lax.axis_index("c")
    s = lax.axis_index("s")
    w = c * NS + s
    ebase = w * EPT

    @pl.loop(0, CH)
    def _(r):
        for j in range(D // 16):
            r0[r, pl.ds(j * 16, 16)] = jnp.zeros((16,), jnp.float32)

    for k in range(RPT // CH):
        pltpu.async_copy(r0, acc.at[pl.ds(s * RPT + k * CH, CH)], semi[0])
    pltpu.async_copy(r0.at[pl.ds(0, RPT % CH)],
                     acc.at[pl.ds(s * RPT + (RPT // CH) * CH, RPT % CH)], semi[0])

    @pl.when(s == 0)
    def _():
        pltpu.async_copy(r0.at[pl.ds(0, TAIL)], acc.at[pl.ds(NS * RPT, TAIL)], semi[0])

    for k in range(RPT // CH):
        pltpu.make_async_copy(r0, acc.at[pl.ds(s * RPT + k * CH, CH)], semi[0]).wait()
    pltpu.make_async_copy(r0.at[pl.ds(0, RPT % CH)],
                          acc.at[pl.ds(s * RPT + (RPT // CH) * CH, RPT % CH)],
                          semi[0]).wait()

    @pl.when(s == 0)
    def _():
        pltpu.make_async_copy(r0.at[pl.ds(0, TAIL)],
                              acc.at[pl.ds(NS * RPT, TAIL)], semi[0]).wait()

    plsc.subcore_barrier()

    def start_idx(i, j):
        pltpu.async_copy(esrc_hbm.at[pl.ds(ebase + i * CH, CH)], sidxb[j], semi[j])
        pltpu.async_copy(edst_hbm.at[pl.ds(ebase + i * CH, CH)], didxb[j], semi[j])

    def wait_idx_start_gather(i, j, b):
        pltpu.make_async_copy(esrc_hbm.at[pl.ds(0, CH)], sidxb[j], semi[j]).wait()
        pltpu.make_async_copy(edst_hbm.at[pl.ds(0, CH)], didxb[j], semi[j]).wait()
        pltpu.async_copy(x_hbm.at[sidxb[j]], bufs[b], semg[b])

    def slot(i):
        b = i % NBUF
        j = i % NIB
        pltpu.make_async_copy(x_hbm.at[sidxb[j]], bufs[b], semg[b]).wait()
        pltpu.async_copy(bufs[b], acc.at[didxb[j]], sems[b], add=True)
        if i + 2 < NIT:
            b2 = (i + 2) % NBUF
            if i >= 1:
                pltpu.make_async_copy(bufs[b2], acc.at[didxb[(i + 2) % NIB]],
                                      sems[b2]).wait()
                if i + NIB - 1 < NIT:
                    start_idx(i + NIB - 1, (i - 1) % NIB)
            wait_idx_start_gather(i + 2, (i + 2) % NIB, b2)

    for j in range(NIB):
        start_idx(j, j)
    for i in range(2):
        wait_idx_start_gather(i, i, i)

    MAIN = (NIT // NIB) * NIB
    @pl.loop(0, MAIN, step=NIB)
    def _(g):
        for b6 in range(NIB):
            i = g + b6
            bb = b6 % NBUF
            pltpu.make_async_copy(x_hbm.at[sidxb[b6]], bufs[bb], semg[bb]).wait()
            pltpu.async_copy(bufs[bb], acc.at[didxb[b6]], sems[bb], add=True)

            @pl.when(i + 2 < NIT)
            def _():
                j2 = (b6 + 2) % NIB
                b2 = (b6 + 2) % NBUF
                jm = (b6 - 1) % NIB

                @pl.when(i >= 1)
                def _():
                    pltpu.make_async_copy(bufs[b2], acc.at[didxb[j2]],
                                          sems[b2]).wait()

                    @pl.when(i + NIB - 1 < NIT)
                    def _():
                        pltpu.async_copy(
                            esrc_hbm.at[pl.ds(ebase + (i + NIB - 1) * CH, CH)],
                            sidxb[jm], semi[jm])
                        pltpu.async_copy(
                            edst_hbm.at[pl.ds(ebase + (i + NIB - 1) * CH, CH)],
                            didxb[jm], semi[jm])

                pltpu.make_async_copy(esrc_hbm.at[pl.ds(0, CH)], sidxb[j2], semi[j2]).wait()
                pltpu.make_async_copy(edst_hbm.at[pl.ds(0, CH)], didxb[j2], semi[j2]).wait()
                pltpu.async_copy(x_hbm.at[sidxb[j2]], bufs[b2], semg[b2])

    for i in range(MAIN, NIT):
        slot(i)

    for cch in range(NIT - NBUF, NIT):
        pltpu.make_async_copy(bufs[cch % NBUF], acc.at[didxb[cch % NIB]],
                              sems[cch % NBUF]).wait()

    plsc.subcore_barrier()

    pltpu.sync_copy(acc.at[pl.ds(s * RPT, RPT)], out_hbm.at[c, pl.ds(s * RPT, RPT)])

    @pl.when(s == 0)
    def _():
        pltpu.sync_copy(acc.at[pl.ds(NS * RPT, TAIL)], out_hbm.at[c, pl.ds(NS * RPT, TAIL)])


_BLK = 400


def _mm_relu_body(p_ref, w_ref, bias_ref, o_ref):
    sm = p_ref[0] + p_ref[1]
    o_ref[...] = jnp.maximum(
        jnp.dot(sm, w_ref[...], preferred_element_type=jnp.float32) + bias_ref[...],
        0.0,
    )


def _mm_relu(p, w, bias):
    grid = (N // _BLK,)
    return pl.pallas_call(
        _mm_relu_body,
        grid=grid,
        in_specs=[
            pl.BlockSpec((NC, _BLK, D), lambda i: (0, i, 0)),
            pl.BlockSpec((D, D), lambda i: (0, 0)),
            pl.BlockSpec((1, D), lambda i: (0, 0)),
        ],
        out_specs=pl.BlockSpec((_BLK, D), lambda i: (i, 0)),
        out_shape=jax.ShapeDtypeStruct((N, D), jnp.float32),
    )(p, w, bias)


def _mm_final_body(p_ref, w2_ref, b2_ref, w3_ref, b3_ref,
                   emb_ref, ge_ref, acc_ref):
    i = pl.program_id(0)
    sm = p_ref[0] + p_ref[1]
    emb_ref[...] = (
        jnp.dot(sm, w2_ref[...], preferred_element_type=jnp.float32) + b2_ref[...]
    )
    csum = jnp.sum(sm, axis=0, keepdims=True)

    @pl.when(i == 0)
    def _():
        acc_ref[...] = csum

    @pl.when(i > 0)
    def _():
        acc_ref[...] = acc_ref[...] + csum

    @pl.when(i == pl.num_programs(0) - 1)
    def _():
        ge_ref[...] = (
            jnp.dot(acc_ref[...], w3_ref[...], preferred_element_type=jnp.float32)
            + float(N) * b3_ref[...]
        )


def _mm_final(p, w2, b2, w3, b3):
    grid = (N // _BLK,)
    return pl.pallas_call(
        _mm_final_body,
        grid=grid,
        in_specs=[
            pl.BlockSpec((NC, _BLK, D), lambda i: (0, i, 0)),
            pl.BlockSpec((D, D), lambda i: (0, 0)),
            pl.BlockSpec((1, D), lambda i: (0, 0)),
            pl.BlockSpec((D, D), lambda i: (0, 0)),
            pl.BlockSpec((1, D), lambda i: (0, 0)),
        ],
        out_specs=[
            pl.BlockSpec((_BLK, D), lambda i: (i, 0)),
            pl.BlockSpec((1, D), lambda i: (0, 0)),
        ],
        out_shape=[
            jax.ShapeDtypeStruct((N, D), jnp.float32),
            jax.ShapeDtypeStruct((1, D), jnp.float32),
        ],
        scratch_shapes=[pltpu.VMEM((1, D), jnp.float32)],
    )(p, w2, b2, w3, b3)


def kernel(x, edge_index, W1, b1, W2, b2, W3, b3):
    esrc = edge_index[0]
    edst = edge_index[1]

    p1 = _edge_scatter(x, esrc, edst)
    h = _mm_relu(p1, W1, b1.reshape(1, D))
    p2 = _edge_scatter(h, esrc, edst)
    embed, graph_embed = _mm_final(
        p2, W2, b2.reshape(1, D), W3, b3.reshape(1, D)
    )
    return (embed, graph_embed)


# speedup vs baseline: 1.2331x; 1.0041x over previous
import functools

import jax
import jax.numpy as jnp
from jax import lax
from jax.experimental import pallas as pl
from jax.experimental.pallas import tpu as pltpu
from jax.experimental.pallas import tpu_sc as plsc

N = 10000
D = 128
E = 320000

NC = 2
NS = 16
NW = NC * NS

EPT = E // NW
CH = 80
NIT = EPT // CH
NBUF = 3
NIB = 6
RPT = 624
TAIL = N - NS * RPT

_mesh = plsc.VectorSubcoreMesh(core_axis_name="c", subcore_axis_name="s")


@functools.partial(
    pl.kernel,
    mesh=_mesh,
    out_type=jax.ShapeDtypeStruct((NC, N, D), jnp.float32),
    scratch_types=(
        [pltpu.VMEM((CH,), jnp.int32) for _ in range(NIB)]
        + [pltpu.VMEM((CH,), jnp.int32) for _ in range(NIB)]
        + [pltpu.VMEM((CH, D), jnp.float32) for _ in range(NBUF)]
        + [pltpu.VMEM_SHARED((N, D), jnp.float32)]
        + [pltpu.SemaphoreType.DMA for _ in range(NIB + 2 * NBUF)]
    ),
)
def _edge_scatter(x_hbm, esrc_hbm, edst_hbm, out_hbm, *rest):
    sidxb = rest[0:NIB]
    didxb = rest[NIB:2 * NIB]
    bufs = rest[2 * NIB:2 * NIB + NBUF]
    acc = rest[2 * NIB + NBUF]
    semi = rest[2 * NIB + NBUF + 1:2 * NIB + NBUF + 1 + NIB]
    semg = rest[2 * NIB + NBUF + 1 + NIB:2 * NIB + NBUF + 1 + NIB + NBUF]
    sems = rest[2 * NIB + NBUF + 1 + NIB + NBUF:]
    r0 = bufs[0]
    c = lax.axis_index("c")
    s = lax.axis_index("s")
    w = c * NS + s
    ebase = w * EPT

    for j in range(NIB):
        pltpu.async_copy(esrc_hbm.at[pl.ds(ebase + j * CH, CH)], sidxb[j], semi[j])
        pltpu.async_copy(edst_hbm.at[pl.ds(ebase + j * CH, CH)], didxb[j], semi[j])

    @pl.loop(0, CH)
    def _(r):
        for j in range(D // 16):
            r0[r, pl.ds(j * 16, 16)] = jnp.zeros((16,), jnp.float32)

    for k in range(RPT // CH):
        pltpu.async_copy(r0, acc.at[pl.ds(s * RPT + k * CH, CH)], semi[0])
    pltpu.async_copy(r0.at[pl.ds(0, RPT % CH)],
                     acc.at[pl.ds(s * RPT + (RPT // CH) * CH, RPT % CH)], semi[0])

    @pl.when(s == 0)
    def _():
        pltpu.async_copy(r0.at[pl.ds(0, TAIL)], acc.at[pl.ds(NS * RPT, TAIL)], semi[0])

    for k in range(RPT // CH):
        pltpu.make_async_copy(r0, acc.at[pl.ds(s * RPT + k * CH, CH)], semi[0]).wait()
    pltpu.make_async_copy(r0.at[pl.ds(0, RPT % CH)],
                          acc.at[pl.ds(s * RPT + (RPT // CH) * CH, RPT % CH)],
                          semi[0]).wait()

    @pl.when(s == 0)
    def _():
        pltpu.make_async_copy(r0.at[pl.ds(0, TAIL)],
                              acc.at[pl.ds(NS * RPT, TAIL)], semi[0]).wait()

    for i0 in range(2):
        pltpu.make_async_copy(esrc_hbm.at[pl.ds(0, CH)], sidxb[i0], semi[i0]).wait()
        pltpu.make_async_copy(edst_hbm.at[pl.ds(0, CH)], didxb[i0], semi[i0]).wait()
        pltpu.async_copy(x_hbm.at[sidxb[i0]], bufs[i0], semg[i0])

    plsc.subcore_barrier()

    def start_idx(i, j):
        pltpu.async_copy(esrc_hbm.at[pl.ds(ebase + i * CH, CH)], sidxb[j], semi[j])
        pltpu.async_copy(edst_hbm.at[pl.ds(ebase + i * CH, CH)], didxb[j], semi[j])

    def wait_idx_start_gather(i, j, b):
        pltpu.make_async_copy(esrc_hbm.at[pl.ds(0, CH)], sidxb[j], semi[j]).wait()
        pltpu.make_async_copy(edst_hbm.at[pl.ds(0, CH)], didxb[j], semi[j]).wait()
        pltpu.async_copy(x_hbm.at[sidxb[j]], bufs[b], semg[b])

    def slot(i):
        b = i % NBUF
        j = i % NIB
        pltpu.make_async_copy(x_hbm.at[sidxb[j]], bufs[b], semg[b]).wait()
        pltpu.async_copy(bufs[b], acc.at[didxb[j]], sems[b], add=True)
        if i + 2 < NIT:
            b2 = (i + 2) % NBUF
            if i >= 1:
                pltpu.make_async_copy(bufs[b2], acc.at[didxb[(i + 2) % NIB]],
                                      sems[b2]).wait()
                if i + NIB - 1 < NIT:
                    start_idx(i + NIB - 1, (i - 1) % NIB)
            wait_idx_start_gather(i + 2, (i + 2) % NIB, b2)

    MAIN = (NIT // NIB) * NIB
    @pl.loop(0, MAIN, step=NIB)
    def _(g):
        for b6 in range(NIB):
            i = g + b6
            bb = b6 % NBUF
            pltpu.make_async_copy(x_hbm.at[sidxb[b6]], bufs[bb], semg[bb]).wait()
            pltpu.async_copy(bufs[bb], acc.at[didxb[b6]], sems[bb], add=True)

            @pl.when(i + 2 < NIT)
            def _():
                j2 = (b6 + 2) % NIB
                b2 = (b6 + 2) % NBUF
                jm = (b6 - 1) % NIB

                @pl.when(i >= 1)
                def _():
                    pltpu.make_async_copy(bufs[b2], acc.at[didxb[j2]],
                                          sems[b2]).wait()

                    @pl.when(i + NIB - 1 < NIT)
                    def _():
                        pltpu.async_copy(
                            esrc_hbm.at[pl.ds(ebase + (i + NIB - 1) * CH, CH)],
                            sidxb[jm], semi[jm])
                        pltpu.async_copy(
                            edst_hbm.at[pl.ds(ebase + (i + NIB - 1) * CH, CH)],
                            didxb[jm], semi[jm])

                pltpu.make_async_copy(esrc_hbm.at[pl.ds(0, CH)], sidxb[j2], semi[j2]).wait()
                pltpu.make_async_copy(edst_hbm.at[pl.ds(0, CH)], didxb[j2], semi[j2]).wait()
                pltpu.async_copy(x_hbm.at[sidxb[j2]], bufs[b2], semg[b2])

    for i in range(MAIN, NIT):
        slot(i)

    for cch in range(NIT - NBUF, NIT):
        pltpu.make_async_copy(bufs[cch % NBUF], acc.at[didxb[cch % NIB]],
                              sems[cch % NBUF]).wait()

    plsc.subcore_barrier()

    pltpu.sync_copy(acc.at[pl.ds(s * RPT, RPT)], out_hbm.at[c, pl.ds(s * RPT, RPT)])

    @pl.when(s == 0)
    def _():
        pltpu.sync_copy(acc.at[pl.ds(NS * RPT, TAIL)], out_hbm.at[c, pl.ds(NS * RPT, TAIL)])


_BLK = 400


def _mm_relu_body(p_ref, w_ref, bias_ref, o_ref):
    sm = p_ref[0] + p_ref[1]
    o_ref[...] = jnp.maximum(
        jnp.dot(sm, w_ref[...], preferred_element_type=jnp.float32) + bias_ref[...],
        0.0,
    )


def _mm_relu(p, w, bias):
    grid = (N // _BLK,)
    return pl.pallas_call(
        _mm_relu_body,
        grid=grid,
        in_specs=[
            pl.BlockSpec((NC, _BLK, D), lambda i: (0, i, 0)),
            pl.BlockSpec((D, D), lambda i: (0, 0)),
            pl.BlockSpec((1, D), lambda i: (0, 0)),
        ],
        out_specs=pl.BlockSpec((_BLK, D), lambda i: (i, 0)),
        out_shape=jax.ShapeDtypeStruct((N, D), jnp.float32),
    )(p, w, bias)


def _mm_final_body(p_ref, w2_ref, b2_ref, w3_ref, b3_ref,
                   emb_ref, ge_ref, acc_ref):
    i = pl.program_id(0)
    sm = p_ref[0] + p_ref[1]
    emb_ref[...] = (
        jnp.dot(sm, w2_ref[...], preferred_element_type=jnp.float32) + b2_ref[...]
    )
    csum = jnp.sum(sm, axis=0, keepdims=True)

    @pl.when(i == 0)
    def _():
        acc_ref[...] = csum

    @pl.when(i > 0)
    def _():
        acc_ref[...] = acc_ref[...] + csum

    @pl.when(i == pl.num_programs(0) - 1)
    def _():
        ge_ref[...] = (
            jnp.dot(acc_ref[...], w3_ref[...], preferred_element_type=jnp.float32)
            + float(N) * b3_ref[...]
        )


def _mm_final(p, w2, b2, w3, b3):
    grid = (N // _BLK,)
    return pl.pallas_call(
        _mm_final_body,
        grid=grid,
        in_specs=[
            pl.BlockSpec((NC, _BLK, D), lambda i: (0, i, 0)),
            pl.BlockSpec((D, D), lambda i: (0, 0)),
            pl.BlockSpec((1, D), lambda i: (0, 0)),
            pl.BlockSpec((D, D), lambda i: (0, 0)),
            pl.BlockSpec((1, D), lambda i: (0, 0)),
        ],
        out_specs=[
            pl.BlockSpec((_BLK, D), lambda i: (i, 0)),
            pl.BlockSpec((1, D), lambda i: (0, 0)),
        ],
        out_shape=[
            jax.ShapeDtypeStruct((N, D), jnp.float32),
            jax.ShapeDtypeStruct((1, D), jnp.float32),
        ],
        scratch_shapes=[pltpu.VMEM((1, D), jnp.float32)],
    )(p, w2, b2, w3, b3)


def kernel(x, edge_index, W1, b1, W2, b2, W3, b3):
    esrc = edge_index[0]
    edst = edge_index[1]

    p1 = _edge_scatter(x, esrc, edst)
    h = _mm_relu(p1, W1, b1.reshape(1, D))
    p2 = _edge_scatter(h, esrc, edst)
    embed, graph_embed = _mm_final(
        p2, W2, b2.reshape(1, D), W3, b3.reshape(1, D)
    )
    return (embed, graph_embed)


# speedup vs baseline: 1.3054x; 1.0586x over previous
import functools

import jax
import jax.numpy as jnp
from jax import lax
from jax.experimental import pallas as pl
from jax.experimental.pallas import tpu as pltpu
from jax.experimental.pallas import tpu_sc as plsc

N = 10000
D = 128
E = 320000

NC = 2
NS = 16
NW = NC * NS

EPT = E // NW
CH = 80
NIT = EPT // CH
NBUF = 3
NIB = 6
RPT = 624
TAIL = N - NS * RPT

_mesh = plsc.VectorSubcoreMesh(core_axis_name="c", subcore_axis_name="s")


@functools.partial(
    pl.kernel,
    mesh=_mesh,
    out_type=jax.ShapeDtypeStruct((NC, N, D), jnp.float32),
    scratch_types=(
        [pltpu.VMEM((CH,), jnp.int32) for _ in range(NIB)]
        + [pltpu.VMEM((CH,), jnp.int32) for _ in range(NIB)]
        + [pltpu.VMEM((CH, D), jnp.float32) for _ in range(NBUF)]
        + [pltpu.VMEM_SHARED((N, D), jnp.float32)]
        + [pltpu.SemaphoreType.DMA for _ in range(NIB + 2 * NBUF)]
    ),
)
def _edge_scatter(x_hbm, esrc_hbm, edst_hbm, out_hbm, *rest):
    sidxb = rest[0:NIB]
    didxb = rest[NIB:2 * NIB]
    bufs = rest[2 * NIB:2 * NIB + NBUF]
    acc = rest[2 * NIB + NBUF]
    semi = rest[2 * NIB + NBUF + 1:2 * NIB + NBUF + 1 + NIB]
    semg = rest[2 * NIB + NBUF + 1 + NIB:2 * NIB + NBUF + 1 + NIB + NBUF]
    sems = rest[2 * NIB + NBUF + 1 + NIB + NBUF:]
    r0 = bufs[0]
    c = lax.axis_index("c")
    s = lax.axis_index("s")
    w = c * NS + s
    ebase = w * EPT

    for j in range(NIB):
        pltpu.async_copy(esrc_hbm.at[pl.ds(ebase + j * CH, CH)], sidxb[j], semi[j])
        pltpu.async_copy(edst_hbm.at[pl.ds(ebase + j * CH, CH)], didxb[j], semi[j])

    @pl.loop(0, CH)
    def _(r):
        for j in range(D // 16):
            r0[r, pl.ds(j * 16, 16)] = jnp.zeros((16,), jnp.float32)

    for k in range(RPT // CH):
        pltpu.async_copy(r0, acc.at[pl.ds(s * RPT + k * CH, CH)], semi[0])
    pltpu.async_copy(r0.at[pl.ds(0, RPT % CH)],
                     acc.at[pl.ds(s * RPT + (RPT // CH) * CH, RPT % CH)], semi[0])

    @pl.when(s == 0)
    def _():
        pltpu.async_copy(r0.at[pl.ds(0, TAIL)], acc.at[pl.ds(NS * RPT, TAIL)], semi[0])

    for k in range(RPT // CH):
        pltpu.make_async_copy(r0, acc.at[pl.ds(s * RPT + k * CH, CH)], semi[0]).wait()
    pltpu.make_async_copy(r0.at[pl.ds(0, RPT % CH)],
                          acc.at[pl.ds(s * RPT + (RPT // CH) * CH, RPT % CH)],
                          semi[0]).wait()

    @pl.when(s == 0)
    def _():
        pltpu.make_async_copy(r0.at[pl.ds(0, TAIL)],
                              acc.at[pl.ds(NS * RPT, TAIL)], semi[0]).wait()

    for i0 in range(2):
        pltpu.make_async_copy(esrc_hbm.at[pl.ds(0, CH)], sidxb[i0], semi[i0]).wait()
        pltpu.make_async_copy(edst_hbm.at[pl.ds(0, CH)], didxb[i0], semi[i0]).wait()
        pltpu.async_copy(x_hbm.at[sidxb[i0]], bufs[i0], semg[i0])

    plsc.subcore_barrier()

    def start_idx(i, j):
        pltpu.async_copy(esrc_hbm.at[pl.ds(ebase + i * CH, CH)], sidxb[j], semi[j])
        pltpu.async_copy(edst_hbm.at[pl.ds(ebase + i * CH, CH)], didxb[j], semi[j])

    def wait_idx_start_gather(i, j, b):
        pltpu.make_async_copy(esrc_hbm.at[pl.ds(0, CH)], sidxb[j], semi[j]).wait()
        pltpu.make_async_copy(edst_hbm.at[pl.ds(0, CH)], didxb[j], semi[j]).wait()
        pltpu.async_copy(x_hbm.at[sidxb[j]], bufs[b], semg[b])

    def slot(i):
        b = i % NBUF
        j = i % NIB
        pltpu.make_async_copy(x_hbm.at[sidxb[j]], bufs[b], semg[b]).wait()
        pltpu.async_copy(bufs[b], acc.at[didxb[j]], sems[b], add=True)
        if i + 2 < NIT:
            b2 = (i + 2) % NBUF
            if i >= 1:
                pltpu.make_async_copy(bufs[b2], acc.at[didxb[(i + 2) % NIB]],
                                      sems[b2]).wait()
                if i + NIB - 1 < NIT:
                    start_idx(i + NIB - 1, (i - 1) % NIB)
            wait_idx_start_gather(i + 2, (i + 2) % NIB, b2)

    MAIN = (NIT // NIB) * NIB
    @pl.loop(0, MAIN, step=NIB)
    def _(g):
        for b6 in range(NIB):
            i = g + b6
            bb = b6 % NBUF
            pltpu.make_async_copy(x_hbm.at[sidxb[b6]], bufs[bb], semg[bb]).wait()
            pltpu.async_copy(bufs[bb], acc.at[didxb[b6]], sems[bb], add=True)

            @pl.when(i + 2 < NIT)
            def _():
                j2 = (b6 + 2) % NIB
                b2 = (b6 + 2) % NBUF
                jm = (b6 - 1) % NIB

                @pl.when(i >= 1)
                def _():
                    pltpu.make_async_copy(bufs[b2], acc.at[didxb[j2]],
                                          sems[b2]).wait()

                    @pl.when(i + NIB - 1 < NIT)
                    def _():
                        pltpu.async_copy(
                            esrc_hbm.at[pl.ds(ebase + (i + NIB - 1) * CH, CH)],
                            sidxb[jm], semi[jm])
                        pltpu.async_copy(
                            edst_hbm.at[pl.ds(ebase + (i + NIB - 1) * CH, CH)],
                            didxb[jm], semi[jm])

                pltpu.make_async_copy(esrc_hbm.at[pl.ds(0, CH)], sidxb[j2], semi[j2]).wait()
                pltpu.make_async_copy(edst_hbm.at[pl.ds(0, CH)], didxb[j2], semi[j2]).wait()
                pltpu.async_copy(x_hbm.at[sidxb[j2]], bufs[b2], semg[b2])

    for i in range(MAIN, NIT):
        slot(i)

    for cch in range(NIT - NBUF, NIT):
        pltpu.make_async_copy(bufs[cch % NBUF], acc.at[didxb[cch % NIB]],
                              sems[cch % NBUF]).wait()

    plsc.subcore_barrier()

    pltpu.sync_copy(acc.at[pl.ds(s * RPT, RPT)], out_hbm.at[c, pl.ds(s * RPT, RPT)])

    @pl.when(s == 0)
    def _():
        pltpu.sync_copy(acc.at[pl.ds(NS * RPT, TAIL)], out_hbm.at[c, pl.ds(NS * RPT, TAIL)])


_BLK = 1000


def _mm_relu_body(p_ref, w_ref, bias_ref, o_ref):
    sm = p_ref[0] + p_ref[1]
    o_ref[...] = jnp.maximum(
        jnp.dot(sm, w_ref[...], preferred_element_type=jnp.float32) + bias_ref[...],
        0.0,
    )


def _mm_relu(p, w, bias):
    grid = (N // _BLK,)
    return pl.pallas_call(
        _mm_relu_body,
        grid=grid,
        in_specs=[
            pl.BlockSpec((NC, _BLK, D), lambda i: (0, i, 0)),
            pl.BlockSpec((D, D), lambda i: (0, 0)),
            pl.BlockSpec((1, D), lambda i: (0, 0)),
        ],
        out_specs=pl.BlockSpec((_BLK, D), lambda i: (i, 0)),
        out_shape=jax.ShapeDtypeStruct((N, D), jnp.float32),
    )(p, w, bias)


def _mm_final_body(p_ref, w2_ref, b2_ref, w3_ref, b3_ref,
                   emb_ref, ge_ref, acc_ref):
    i = pl.program_id(0)
    sm = p_ref[0] + p_ref[1]
    emb_ref[...] = (
        jnp.dot(sm, w2_ref[...], preferred_element_type=jnp.float32) + b2_ref[...]
    )
    csum = jnp.sum(sm, axis=0, keepdims=True)

    @pl.when(i == 0)
    def _():
        acc_ref[...] = csum

    @pl.when(i > 0)
    def _():
        acc_ref[...] = acc_ref[...] + csum

    @pl.when(i == pl.num_programs(0) - 1)
    def _():
        ge_ref[...] = (
            jnp.dot(acc_ref[...], w3_ref[...], preferred_element_type=jnp.float32)
            + float(N) * b3_ref[...]
        )


def _mm_final(p, w2, b2, w3, b3):
    grid = (N // _BLK,)
    return pl.pallas_call(
        _mm_final_body,
        grid=grid,
        in_specs=[
            pl.BlockSpec((NC, _BLK, D), lambda i: (0, i, 0)),
            pl.BlockSpec((D, D), lambda i: (0, 0)),
            pl.BlockSpec((1, D), lambda i: (0, 0)),
            pl.BlockSpec((D, D), lambda i: (0, 0)),
            pl.BlockSpec((1, D), lambda i: (0, 0)),
        ],
        out_specs=[
            pl.BlockSpec((_BLK, D), lambda i: (i, 0)),
            pl.BlockSpec((1, D), lambda i: (0, 0)),
        ],
        out_shape=[
            jax.ShapeDtypeStruct((N, D), jnp.float32),
            jax.ShapeDtypeStruct((1, D), jnp.float32),
        ],
        scratch_shapes=[pltpu.VMEM((1, D), jnp.float32)],
    )(p, w2, b2, w3, b3)


def kernel(x, edge_index, W1, b1, W2, b2, W3, b3):
    esrc = edge_index[0]
    edst = edge_index[1]

    p1 = _edge_scatter(x, esrc, edst)
    h = _mm_relu(p1, W1, b1.reshape(1, D))
    p2 = _edge_scatter(h, esrc, edst)
    embed, graph_embed = _mm_final(
        p2, W2, b2.reshape(1, D), W3, b3.reshape(1, D)
    )
    return (embed, graph_embed)
